# baseline (device time: 15481 ns/iter reference)
import functools
import math

import jax
import jax.numpy as jnp
from jax import lax
from jax.experimental import pallas as pl
from jax.experimental.pallas import tpu as pltpu

N_DEV = 4


def kernel(q, k, v):
    s_per, d = q.shape
    scale = 1.0 / math.sqrt(d)

    def body(q_ref, k_ref, v_ref, out_ref, kv_buf, send_sems, recv_sems):
        my = lax.axis_index("i")
        left = lax.rem(my + N_DEV - 1, N_DEV)
        right = lax.rem(my + 1, N_DEV)

        barrier_sem = pltpu.get_barrier_semaphore()
        for nbr in (left, right):
            pl.semaphore_signal(
                barrier_sem, inc=1,
                device_id=(nbr,), device_id_type=pl.DeviceIdType.MESH,
            )
        pl.semaphore_wait(barrier_sem, 2)

        q_bf = (q_ref[:, :] * scale).astype(jnp.bfloat16)
        kv_buf[0, 0, :, :] = k_ref[:, :].astype(jnp.bfloat16)
        kv_buf[0, 1, :, :] = v_ref[:, :].astype(jnp.bfloat16)

        m = jnp.full((s_per, 1), -jnp.inf, dtype=jnp.float32)
        l = jnp.zeros((s_per, 1), dtype=jnp.float32)
        acc = jnp.zeros((s_per, d), dtype=jnp.float32)

        rdmas = []
        for h in range(N_DEV):
            if h > 0:
                rdmas[h - 1].wait_recv()
            if h < N_DEV - 1:
                rdma = pltpu.make_async_remote_copy(
                    src_ref=kv_buf.at[h],
                    dst_ref=kv_buf.at[h + 1],
                    send_sem=send_sems.at[h],
                    recv_sem=recv_sems.at[h],
                    device_id=(right,),
                    device_id_type=pl.DeviceIdType.MESH,
                )
                rdma.start()
                rdmas.append(rdma)

            k_h = kv_buf[h, 0, :, :]
            v_h = kv_buf[h, 1, :, :]
            s = lax.dot_general(
                q_bf, k_h,
                (((1,), (1,)), ((), ())),
                preferred_element_type=jnp.float32,
            )
            m_new = jnp.maximum(m, jnp.max(s, axis=1, keepdims=True))
            p = jnp.exp(s - m_new)
            alpha = jnp.exp(m - m_new)
            l = l * alpha + jnp.sum(p, axis=1, keepdims=True)
            pv = lax.dot_general(
                p.astype(jnp.bfloat16), v_h,
                (((1,), (0,)), ((), ())),
                preferred_element_type=jnp.float32,
            )
            acc = acc * alpha + pv
            m = m_new

        for rdma in rdmas:
            rdma.wait_send()

        out_ref[:, :] = acc / l

    return pl.pallas_call(
        body,
        out_shape=jax.ShapeDtypeStruct((s_per, d), jnp.float32),
        in_specs=[
            pl.BlockSpec(memory_space=pltpu.VMEM),
            pl.BlockSpec(memory_space=pltpu.VMEM),
            pl.BlockSpec(memory_space=pltpu.VMEM),
        ],
        out_specs=pl.BlockSpec(memory_space=pltpu.VMEM),
        scratch_shapes=[
            pltpu.VMEM((N_DEV, 2, s_per, d), jnp.bfloat16),
            pltpu.SemaphoreType.DMA((N_DEV - 1,)),
            pltpu.SemaphoreType.DMA((N_DEV - 1,)),
        ],
        compiler_params=pltpu.CompilerParams(collective_id=0),
    )(q, k, v)


# device time: 11297 ns/iter; 1.3704x vs baseline; 1.3704x over previous
import functools
import math

import jax
import jax.numpy as jnp
from jax import lax
from jax.experimental import pallas as pl
from jax.experimental.pallas import tpu as pltpu

N_DEV = 4


def kernel(q, k, v):
    s_per, d = q.shape
    scale = 1.0 / math.sqrt(d)

    def body(q_ref, k_ref, v_ref, out_ref, kv_buf, send_sems, recv_sems):
        my = lax.axis_index("i")

        barrier_sem = pltpu.get_barrier_semaphore()
        for o in (1, 2, 3):
            nbr = lax.rem(my + o, N_DEV)
            pl.semaphore_signal(
                barrier_sem, inc=1,
                device_id=(nbr,), device_id_type=pl.DeviceIdType.MESH,
            )
        pl.semaphore_wait(barrier_sem, 3)

        q_bf = (q_ref[:, :] * scale).astype(jnp.bfloat16)
        kv_buf[0, 0, :, :] = k_ref[:, :].astype(jnp.bfloat16)
        kv_buf[0, 1, :, :] = v_ref[:, :].astype(jnp.bfloat16)

        rdmas = {}
        for o in (1, 2, 3):
            tgt = lax.rem(my + o, N_DEV)
            rdma = pltpu.make_async_remote_copy(
                src_ref=kv_buf.at[0],
                dst_ref=kv_buf.at[o],
                send_sem=send_sems.at[o - 1],
                recv_sem=recv_sems.at[o - 1],
                device_id=(tgt,),
                device_id_type=pl.DeviceIdType.MESH,
            )
            rdma.start()
            rdmas[o] = rdma

        m = jnp.full((s_per, 1), -jnp.inf, dtype=jnp.float32)
        l = jnp.zeros((s_per, 1), dtype=jnp.float32)
        acc = jnp.zeros((s_per, d), dtype=jnp.float32)

        def process(slot, m, l, acc):
            k_h = kv_buf[slot, 0, :, :]
            v_h = kv_buf[slot, 1, :, :]
            s = lax.dot_general(
                q_bf, k_h,
                (((1,), (1,)), ((), ())),
                preferred_element_type=jnp.float32,
            )
            m_new = jnp.maximum(m, jnp.max(s, axis=1, keepdims=True))
            p = jnp.exp(s - m_new)
            alpha = jnp.exp(m - m_new)
            l = l * alpha + jnp.sum(p, axis=1, keepdims=True)
            pv = lax.dot_general(
                p.astype(jnp.bfloat16), v_h,
                (((1,), (0,)), ((), ())),
                preferred_element_type=jnp.float32,
            )
            acc = acc * alpha + pv
            return m_new, l, acc

        m, l, acc = process(0, m, l, acc)
        for o in (1, 3, 2):
            rdmas[o].wait_recv()
            m, l, acc = process(o, m, l, acc)

        for o in (1, 2, 3):
            rdmas[o].wait_send()

        out_ref[:, :] = acc / l

    return pl.pallas_call(
        body,
        out_shape=jax.ShapeDtypeStruct((s_per, d), jnp.float32),
        in_specs=[
            pl.BlockSpec(memory_space=pltpu.VMEM),
            pl.BlockSpec(memory_space=pltpu.VMEM),
            pl.BlockSpec(memory_space=pltpu.VMEM),
        ],
        out_specs=pl.BlockSpec(memory_space=pltpu.VMEM),
        scratch_shapes=[
            pltpu.VMEM((N_DEV, 2, s_per, d), jnp.bfloat16),
            pltpu.SemaphoreType.DMA((N_DEV - 1,)),
            pltpu.SemaphoreType.DMA((N_DEV - 1,)),
        ],
        compiler_params=pltpu.CompilerParams(collective_id=0),
    )(q, k, v)


# device time: 10932 ns/iter; 1.4161x vs baseline; 1.0334x over previous
import math

import jax
import jax.numpy as jnp
from jax import lax
from jax.experimental import pallas as pl
from jax.experimental.pallas import tpu as pltpu

N_DEV = 4


def kernel(q, k, v):
    s_per, d = q.shape
    scale = 1.0 / math.sqrt(d)

    def body(q_ref, k_ref, v_ref, out_ref, kv_buf, send_sems, recv_sems,
             entry_sems):
        my = lax.axis_index("i")

        barrier_sem = pltpu.get_barrier_semaphore()
        pl.semaphore_signal(barrier_sem, inc=1)
        pl.semaphore_wait(barrier_sem, 1)

        for o in (1, 2, 3):
            nbr = lax.rem(my + o, N_DEV)
            pl.semaphore_signal(
                entry_sems.at[3 - o], inc=1,
                device_id=(nbr,), device_id_type=pl.DeviceIdType.MESH,
            )

        kv_buf[0, 0, :, :] = k_ref[:, :].astype(jnp.bfloat16)
        kv_buf[0, 1, :, :] = v_ref[:, :].astype(jnp.bfloat16)

        rdmas = {}
        for o in (1, 3, 2):
            tgt = lax.rem(my + o, N_DEV)
            pl.semaphore_wait(entry_sems.at[o - 1], 1)
            rdma = pltpu.make_async_remote_copy(
                src_ref=kv_buf.at[0],
                dst_ref=kv_buf.at[o],
                send_sem=send_sems.at[o - 1],
                recv_sem=recv_sems.at[o - 1],
                device_id=(tgt,),
                device_id_type=pl.DeviceIdType.MESH,
            )
            rdma.start()
            rdmas[o] = rdma

        q_bf = (q_ref[:, :] * scale).astype(jnp.bfloat16)

        def process(slot, m, l, acc):
            k_h = kv_buf[slot, 0, :, :]
            v_h = kv_buf[slot, 1, :, :]
            s = lax.dot_general(
                q_bf, k_h,
                (((1,), (1,)), ((), ())),
                preferred_element_type=jnp.float32,
            )
            m_new = jnp.maximum(m, jnp.max(s, axis=1, keepdims=True))
            p = jnp.exp(s - m_new)
            alpha = jnp.exp(m - m_new)
            l = l * alpha + jnp.sum(p, axis=1, keepdims=True)
            pv = lax.dot_general(
                p.astype(jnp.bfloat16), v_h,
                (((1,), (0,)), ((), ())),
                preferred_element_type=jnp.float32,
            )
            acc = acc * alpha + pv
            return m_new, l, acc

        m = jnp.full((s_per, 1), -jnp.inf, dtype=jnp.float32)
        l = jnp.zeros((s_per, 1), dtype=jnp.float32)
        acc = jnp.zeros((s_per, d), dtype=jnp.float32)
        m, l, acc = process(0, m, l, acc)

        for o in (1, 3, 2):
            rdmas[o].wait_recv()
            m, l, acc = process(o, m, l, acc)

        for o in (1, 3, 2):
            rdmas[o].wait_send()

        out_ref[:, :] = acc * (1.0 / l)

    return pl.pallas_call(
        body,
        out_shape=jax.ShapeDtypeStruct((s_per, d), jnp.float32),
        in_specs=[
            pl.BlockSpec(memory_space=pltpu.VMEM),
            pl.BlockSpec(memory_space=pltpu.VMEM),
            pl.BlockSpec(memory_space=pltpu.VMEM),
        ],
        out_specs=pl.BlockSpec(memory_space=pltpu.VMEM),
        scratch_shapes=[
            pltpu.VMEM((N_DEV, 2, s_per, d), jnp.bfloat16),
            pltpu.SemaphoreType.DMA((N_DEV - 1,)),
            pltpu.SemaphoreType.DMA((N_DEV - 1,)),
            pltpu.SemaphoreType.REGULAR((N_DEV - 1,)),
        ],
        compiler_params=pltpu.CompilerParams(collective_id=0),
    )(q, k, v)
